# baseline (speedup 1.0000x reference)
import functools

import jax
import jax.numpy as jnp
from jax import lax
from jax.experimental import pallas as pl
from jax.experimental.pallas import tpu as pltpu
from jax.experimental.pallas import tpu_sc as plsc

_B = 4096
_LX = 50
_LL = 20
_D = 64
_NC, _NS = 2, 16
_NW = _NC * _NS
_RPW = _B // _NW
_XBLK = _LX // 2
_LBLK = _LL // 2
_XPAD = 32
_LPAD = 16
_XRC = 2
_LRC = 4
_NBUF = 4


def _row_gather_phase(eidx_hbm, oidx_hbm, table, out_hbm, eidx_v, oidx_v,
                      erows, orows, gsems, wsems, half, pad, rpc, wid):
    b0 = wid * _RPW
    n = rpc * pad
    pltpu.sync_copy(eidx_hbm.at[pl.ds(b0 * pad, _RPW * pad)], eidx_v)
    pltpu.sync_copy(oidx_hbm.at[pl.ds(b0 * pad, _RPW * pad)], oidx_v)

    def g_start(c, s):
        pltpu.make_async_copy(
            table.at[eidx_v.at[pl.ds(c * n, n)]],
            erows.at[s, pl.ds(0, n)], gsems[s]).start()
        pltpu.make_async_copy(
            table.at[oidx_v.at[pl.ds(c * n, n)]],
            orows.at[s, pl.ds(0, n)], gsems[s]).start()

    def g_wait(s):
        for rows in (erows, orows):
            pltpu.make_async_copy(
                table.at[eidx_v.at[pl.ds(0, n)]],
                rows.at[s, pl.ds(0, n)], gsems[s]).wait()

    def w_start(c, s):
        for r in range(rpc):
            b = b0 + c * rpc + r
            pltpu.make_async_copy(
                erows.at[s, pl.ds(r * pad, half)],
                out_hbm.at[:, b, pl.ds(0, _D)], wsems[s]).start()
            pltpu.make_async_copy(
                orows.at[s, pl.ds(r * pad, half)],
                out_hbm.at[:, b, pl.ds(_D, _D)], wsems[s]).start()

    def w_wait(s):
        for r in range(rpc):
            for rows in (erows, orows):
                pltpu.make_async_copy(
                    rows.at[s, pl.ds(r * pad, half)],
                    out_hbm.at[:, b0, pl.ds(0, _D)], wsems[s]).wait()

    for s in range(_NBUF):
        g_start(s, s)

    nchunks = _RPW // rpc
    ngroups = nchunks // _NBUF

    def group(g, carry):
        for s in range(_NBUF):
            c = g * _NBUF + s
            g_wait(s)
            w_start(c, s)
            w_wait(s)
            g_start(c + _NBUF, s)
        return carry

    lax.fori_loop(0, ngroups - 1, group, 0)
    for s in range(_NBUF):
        c = (ngroups - 1) * _NBUF + s
        g_wait(s)
        w_start(c, s)
        w_wait(s)


def _sc_gather(xe_i, xo_i, le_i, lo_i, wtab, ltab):
    mesh = plsc.VectorSubcoreMesh(
        core_axis_name="c", subcore_axis_name="s",
        num_cores=_NC, num_subcores=_NS,
    )

    @functools.partial(
        pl.kernel,
        out_type=[
            jax.ShapeDtypeStruct((_XBLK, _B, 128), jnp.float32),
            jax.ShapeDtypeStruct((_LBLK, _B, 128), jnp.float32),
        ],
        mesh=mesh,
        scratch_types=[
            pltpu.VMEM((_RPW * _XPAD,), jnp.int32),
            pltpu.VMEM((_RPW * _XPAD,), jnp.int32),
            pltpu.VMEM((_RPW * _LPAD,), jnp.int32),
            pltpu.VMEM((_RPW * _LPAD,), jnp.int32),
            pltpu.VMEM((_NBUF, _XRC * _XPAD, _D), jnp.float32),
            pltpu.VMEM((_NBUF, _XRC * _XPAD, _D), jnp.float32),
        ] + [pltpu.SemaphoreType.DMA] * (2 * _NBUF),
        compiler_params=pltpu.CompilerParams(use_tc_tiling_on_sc=False),
    )
    def run(xe_hbm, xo_hbm, le_hbm, lo_hbm, wtab_hbm, ltab_hbm,
            xeo_hbm, leo_hbm, xei_v, xoi_v, lei_v, loi_v, erows, orows,
            *sems):
        wid = lax.axis_index("s") * _NC + lax.axis_index("c")
        gs, ws = sems[:_NBUF], sems[_NBUF:]
        _row_gather_phase(xe_hbm, xo_hbm, wtab_hbm, xeo_hbm, xei_v, xoi_v,
                          erows, orows, gs, ws, _XBLK, _XPAD, _XRC, wid)
        _row_gather_phase(le_hbm, lo_hbm, ltab_hbm, leo_hbm, lei_v, loi_v,
                          erows, orows, gs, ws, _LBLK, _LPAD, _LRC, wid)

    return run(xe_i, xo_i, le_i, lo_i, wtab, ltab)


def _mlp_body(xe_ref, le_ref, w1x_ref, w1l_ref, b1_ref, w2_ref, b2_ref,
              w3_ref, b3_ref, out_ref):
    x = jnp.concatenate([xe_ref[j] for j in range(_XBLK)], axis=1)
    l = jnp.concatenate([le_ref[j] for j in range(_LBLK)], axis=1)
    h = jnp.dot(x, w1x_ref[...], preferred_element_type=jnp.float32)
    h = h + jnp.dot(l, w1l_ref[...], preferred_element_type=jnp.float32)
    h = jnp.maximum(h + b1_ref[...], 0.0)
    h = jnp.maximum(
        jnp.dot(h, w2_ref[...], preferred_element_type=jnp.float32) + b2_ref[...], 0.0)
    out_ref[...] = jnp.dot(h, w3_ref[...], preferred_element_type=jnp.float32) + b3_ref[...]


def _mlp(xe, le, w1x, w1l, b1, w2, b2, w3, b3):
    bt = 256
    return pl.pallas_call(
        _mlp_body,
        grid=(_B // bt,),
        in_specs=[
            pl.BlockSpec((_XBLK, bt, 128), lambda i: (0, i, 0)),
            pl.BlockSpec((_LBLK, bt, 128), lambda i: (0, i, 0)),
            pl.BlockSpec((_LX * _D, 1024), lambda i: (0, 0)),
            pl.BlockSpec((_LL * _D, 1024), lambda i: (0, 0)),
            pl.BlockSpec((1, 1024), lambda i: (0, 0)),
            pl.BlockSpec((1024, 512), lambda i: (0, 0)),
            pl.BlockSpec((1, 512), lambda i: (0, 0)),
            pl.BlockSpec((512, _LX), lambda i: (0, 0)),
            pl.BlockSpec((1, _LX), lambda i: (0, 0)),
        ],
        out_specs=pl.BlockSpec((bt, _LX), lambda i: (i, 0)),
        out_shape=jax.ShapeDtypeStruct((_B, _LX), jnp.float32),
    )(xe, le, w1x, w1l, b1, w2, b2, w3, b3)


def _pad_idx(a, pad):
    return jnp.pad(a, ((0, 0), (0, pad - a.shape[1]))).reshape(-1)


def kernel(x, lemma, word_emb, lemma_emb, W1, b1, W2, b2, W3, b3):
    xe, le = _sc_gather(
        _pad_idx(x[:, 0::2], _XPAD), _pad_idx(x[:, 1::2], _XPAD),
        _pad_idx(lemma[:, 0::2], _LPAD), _pad_idx(lemma[:, 1::2], _LPAD),
        word_emb, lemma_emb)
    return _mlp(xe, le, W1[: _LX * _D], W1[_LX * _D:], b1.reshape(1, -1),
                W2, b2.reshape(1, -1), W3, b3.reshape(1, -1))


# speedup vs baseline: 3.5432x; 3.5432x over previous
import functools

import jax
import jax.numpy as jnp
from jax import lax
from jax.experimental import pallas as pl
from jax.experimental.pallas import tpu as pltpu
from jax.experimental.pallas import tpu_sc as plsc

_B = 4096
_LX = 50
_LL = 20
_D = 64
_NC, _NS = 2, 16
_NW = _NC * _NS
_CH = 128
_NBUF = 5
_XBLK = _LX // 2
_LBLK = _LL // 2

_XCH = _B * _LX // (_NW * _CH)
_LCH = _B * _LL // (_NW * _CH)


def _gather_phase(idx_hbm, table, out_hbm, idx_v, rows, gsems, wsems, nch, wid):
    pltpu.sync_copy(idx_hbm.at[wid], idx_v.at[pl.ds(0, nch)])
    base = wid * nch * _CH

    def g_start(c, b):
        pltpu.make_async_copy(table.at[idx_v.at[c]], rows.at[b], gsems[b]).start()

    def g_wait(b):
        pltpu.make_async_copy(table.at[idx_v.at[0]], rows.at[b], gsems[b]).wait()

    def w_start(c, b):
        pltpu.make_async_copy(
            rows.at[b], out_hbm.at[pl.ds(base + c * _CH, _CH)], wsems[b]
        ).start()

    def w_wait(b):
        pltpu.make_async_copy(
            rows.at[b], out_hbm.at[pl.ds(base, _CH)], wsems[b]
        ).wait()

    for b in range(_NBUF):
        g_start(b, b)

    ngroups = nch // _NBUF

    def group(g, carry):
        for b in range(_NBUF):
            c = g * _NBUF + b
            g_wait(b)
            w_start(c, b)
            w_wait(b)
            g_start(c + _NBUF, b)
        return carry

    lax.fori_loop(0, ngroups - 1, group, 0)
    for b in range(_NBUF):
        c = (ngroups - 1) * _NBUF + b
        g_wait(b)
        w_start(c, b)
        w_wait(b)


def _sc_gather(xi, li, wtab, ltab):
    mesh = plsc.VectorSubcoreMesh(
        core_axis_name="c", subcore_axis_name="s",
        num_cores=_NC, num_subcores=_NS,
    )

    @functools.partial(
        pl.kernel,
        out_type=[
            jax.ShapeDtypeStruct((_B * _LX, _D), jnp.float32),
            jax.ShapeDtypeStruct((_B * _LL, _D), jnp.float32),
        ],
        mesh=mesh,
        scratch_types=[
            pltpu.VMEM((_XCH, _CH), jnp.int32),
            pltpu.VMEM((_NBUF, _CH, _D), jnp.float32),
        ] + [pltpu.SemaphoreType.DMA] * (2 * _NBUF),
        compiler_params=pltpu.CompilerParams(use_tc_tiling_on_sc=False),
    )
    def run(xi_hbm, li_hbm, wtab_hbm, ltab_hbm, xe_hbm, le_hbm, idx_v, rows, *sems):
        wid = lax.axis_index("s") * _NC + lax.axis_index("c")
        gs, ws = sems[:_NBUF], sems[_NBUF:]
        _gather_phase(xi_hbm, wtab_hbm, xe_hbm, idx_v, rows, gs, ws, _XCH, wid)
        _gather_phase(li_hbm, ltab_hbm, le_hbm, idx_v, rows, gs, ws, _LCH, wid)

    return run(xi, li, wtab, ltab)


def _regroup(v, nblk, bt):
    v = v.reshape(bt // 8, nblk, 8, 128)
    return jnp.concatenate(
        [v[:, j].reshape(bt, 128) for j in range(nblk)], axis=1)


def _mlp_body(xe_ref, le_ref, w1x_ref, w1l_ref, b1_ref, w2_ref, b2_ref,
              w3_ref, b3_ref, out_ref):
    bt = out_ref.shape[0]
    x = _regroup(xe_ref[...], _XBLK, bt)
    l = _regroup(le_ref[...], _LBLK, bt)
    h = jnp.dot(x, w1x_ref[...], preferred_element_type=jnp.float32)
    h = h + jnp.dot(l, w1l_ref[...], preferred_element_type=jnp.float32)
    h = jnp.maximum(h + b1_ref[...], 0.0)
    h = jnp.maximum(
        jnp.dot(h, w2_ref[...], preferred_element_type=jnp.float32) + b2_ref[...], 0.0)
    out_ref[...] = jnp.dot(h, w3_ref[...], preferred_element_type=jnp.float32) + b3_ref[...]


def _mlp(xe, le, w1x, w1l, b1, w2, b2, w3, b3):
    bt = 256
    return pl.pallas_call(
        _mlp_body,
        grid=(_B // bt,),
        in_specs=[
            pl.BlockSpec((bt * _XBLK, 128), lambda i: (i, 0)),
            pl.BlockSpec((bt * _LBLK, 128), lambda i: (i, 0)),
            pl.BlockSpec((_LX * _D, 1024), lambda i: (0, 0)),
            pl.BlockSpec((_LL * _D, 1024), lambda i: (0, 0)),
            pl.BlockSpec((1, 1024), lambda i: (0, 0)),
            pl.BlockSpec((1024, 512), lambda i: (0, 0)),
            pl.BlockSpec((1, 512), lambda i: (0, 0)),
            pl.BlockSpec((512, _LX), lambda i: (0, 0)),
            pl.BlockSpec((1, _LX), lambda i: (0, 0)),
        ],
        out_specs=pl.BlockSpec((bt, _LX), lambda i: (i, 0)),
        out_shape=jax.ShapeDtypeStruct((_B, _LX), jnp.float32),
    )(xe, le, w1x, w1l, b1, w2, b2, w3, b3)


def _tile_order(idx, nblk):
    return idx.reshape(_B // 8, 8, nblk, 2).transpose(0, 2, 1, 3).reshape(-1)


def kernel(x, lemma, word_emb, lemma_emb, W1, b1, W2, b2, W3, b3):
    xi = _tile_order(x, _XBLK).reshape(_NW, _XCH, _CH)
    li = _tile_order(lemma, _LBLK).reshape(_NW, _LCH, _CH)
    xe, le = _sc_gather(xi, li, word_emb, lemma_emb)
    xe = xe.reshape(_B * _LX // 2, 2 * _D)
    le = le.reshape(_B * _LL // 2, 2 * _D)
    return _mlp(xe, le, W1[: _LX * _D], W1[_LX * _D:], b1.reshape(1, -1),
                W2, b2.reshape(1, -1), W3, b3.reshape(1, -1))


# speedup vs baseline: 3.8348x; 1.0823x over previous
import functools

import jax
import jax.numpy as jnp
from jax import lax
from jax.experimental import pallas as pl
from jax.experimental.pallas import tpu as pltpu
from jax.experimental.pallas import tpu_sc as plsc

_B = 4096
_LX = 50
_LL = 20
_D = 64
_NC, _NS = 2, 16
_NW = _NC * _NS
_CH = 128
_NBUF = 5
_XBLK = _LX // 2
_LBLK = _LL // 2

_XCH = _B * _LX // (_NW * _CH)
_LCH = _B * _LL // (_NW * _CH)


def _gather_phase(idx_hbm, table, out_hbm, idx_v, rows, gsems, wsems, nch, wid):
    pltpu.sync_copy(idx_hbm.at[wid], idx_v.at[pl.ds(0, nch)])
    base = wid * nch * _CH

    def g_start(c, b):
        pltpu.make_async_copy(table.at[idx_v.at[c]], rows.at[b], gsems[b]).start()

    def g_wait(b):
        pltpu.make_async_copy(table.at[idx_v.at[0]], rows.at[b], gsems[b]).wait()

    def w_start(c, b):
        pltpu.make_async_copy(
            rows.at[b], out_hbm.at[pl.ds(base + c * _CH, _CH)], wsems[b]
        ).start()

    def w_wait(b):
        pltpu.make_async_copy(
            rows.at[b], out_hbm.at[pl.ds(base, _CH)], wsems[b]
        ).wait()

    for b in range(_NBUF):
        g_start(b, b)

    ngroups = nch // _NBUF

    def group(g, carry):
        for b in range(_NBUF):
            c = g * _NBUF + b
            g_wait(b)
            w_start(c, b)
            w_wait(b)
            g_start(c + _NBUF, b)
        return carry

    lax.fori_loop(0, ngroups - 1, group, 0)
    for b in range(_NBUF):
        c = (ngroups - 1) * _NBUF + b
        g_wait(b)
        w_start(c, b)
        w_wait(b)


def _sc_gather(xi, li, wtab, ltab):
    mesh = plsc.VectorSubcoreMesh(
        core_axis_name="c", subcore_axis_name="s",
        num_cores=_NC, num_subcores=_NS,
    )

    @functools.partial(
        pl.kernel,
        out_type=[
            jax.ShapeDtypeStruct((_B * _LX, _D), jnp.float32),
            jax.ShapeDtypeStruct((_B * _LL, _D), jnp.float32),
        ],
        mesh=mesh,
        scratch_types=[
            pltpu.VMEM((_XCH, _CH), jnp.int32),
            pltpu.VMEM((_NBUF, _CH, _D), jnp.float32),
        ] + [pltpu.SemaphoreType.DMA] * (2 * _NBUF),
        compiler_params=pltpu.CompilerParams(use_tc_tiling_on_sc=False),
    )
    def run(xi_hbm, li_hbm, wtab_hbm, ltab_hbm, xe_hbm, le_hbm, idx_v, rows, *sems):
        wid = lax.axis_index("s") * _NC + lax.axis_index("c")
        gs, ws = sems[:_NBUF], sems[_NBUF:]
        _gather_phase(xi_hbm, wtab_hbm, xe_hbm, idx_v, rows, gs, ws, _XCH, wid)
        _gather_phase(li_hbm, ltab_hbm, le_hbm, idx_v, rows, gs, ws, _LCH, wid)

    return run(xi, li, wtab, ltab)


def _regroup(v, nblk, bt):
    v = v.reshape(bt // 8, nblk, 8, 128)
    return jnp.concatenate(
        [v[:, j].reshape(bt, 128) for j in range(nblk)], axis=1)


def _mlp_body(xe_ref, le_ref, w1x_ref, w1l_ref, b1_ref, w2_ref, b2_ref,
              w3_ref, b3_ref, out_ref):
    bt = out_ref.shape[0]
    x = _regroup(xe_ref[...], _XBLK, bt)
    l = _regroup(le_ref[...], _LBLK, bt)
    h = jnp.dot(x, w1x_ref[...], preferred_element_type=jnp.float32)
    h = h + jnp.dot(l, w1l_ref[...], preferred_element_type=jnp.float32)
    h = jnp.maximum(h + b1_ref[...], 0.0)
    h = jnp.maximum(
        jnp.dot(h, w2_ref[...], preferred_element_type=jnp.float32) + b2_ref[...], 0.0)
    out_ref[...] = jnp.dot(h, w3_ref[...], preferred_element_type=jnp.float32) + b3_ref[...]


def _mlp(xe, le, w1x, w1l, b1, w2, b2, w3, b3):
    bt = 256
    return pl.pallas_call(
        _mlp_body,
        grid=(_B // bt,),
        in_specs=[
            pl.BlockSpec((bt * _XBLK, 128), lambda i: (i, 0)),
            pl.BlockSpec((bt * _LBLK, 128), lambda i: (i, 0)),
            pl.BlockSpec((_LX * _D, 1024), lambda i: (0, 0)),
            pl.BlockSpec((_LL * _D, 1024), lambda i: (0, 0)),
            pl.BlockSpec((1, 1024), lambda i: (0, 0)),
            pl.BlockSpec((1024, 512), lambda i: (0, 0)),
            pl.BlockSpec((1, 512), lambda i: (0, 0)),
            pl.BlockSpec((512, _LX), lambda i: (0, 0)),
            pl.BlockSpec((1, _LX), lambda i: (0, 0)),
        ],
        out_specs=pl.BlockSpec((bt, _LX), lambda i: (i, 0)),
        out_shape=jax.ShapeDtypeStruct((_B, _LX), jnp.float32),
    )(xe, le, w1x, w1l, b1, w2, b2, w3, b3)


def _tile_order(idx, nblk):
    return idx.reshape(_B // 8, 8, nblk, 2).transpose(0, 2, 1, 3).reshape(-1)


def _pad_minor(t):
    return jnp.pad(t, ((0, 0), (0, _D))).reshape(2 * t.shape[0], _D)


def kernel(x, lemma, word_emb, lemma_emb, W1, b1, W2, b2, W3, b3):
    xi = (2 * _tile_order(x, _XBLK)).reshape(_NW, _XCH, _CH)
    li = (2 * _tile_order(lemma, _LBLK)).reshape(_NW, _LCH, _CH)
    xe, le = _sc_gather(xi, li, _pad_minor(word_emb), _pad_minor(lemma_emb))
    xe = xe.reshape(_B * _LX // 2, 2 * _D)
    le = le.reshape(_B * _LL // 2, 2 * _D)
    return _mlp(xe, le, W1[: _LX * _D], W1[_LX * _D:], b1.reshape(1, -1),
                W2, b2.reshape(1, -1), W3, b3.reshape(1, -1))


# speedup vs baseline: 4.3004x; 1.1214x over previous
import functools

import jax
import jax.numpy as jnp
from jax import lax
from jax.experimental import pallas as pl
from jax.experimental.pallas import tpu as pltpu
from jax.experimental.pallas import tpu_sc as plsc

_B = 4096
_LX = 50
_LL = 20
_D = 64
_NC, _NS = 2, 16
_NW = _NC * _NS
_CH = 128
_NBUF = 5
_XBLK = _LX // 2
_LBLK = _LL // 2

_XCH = _B * _LX // (_NW * _CH)
_LCH = _B * _LL // (_NW * _CH)


def _gather_phase(idx_hbm, table, out_hbm, idx_v, rows, gsems, wsems, nch, wid):
    pltpu.sync_copy(idx_hbm.at[wid], idx_v.at[pl.ds(0, nch)])
    base = wid * nch * _CH

    def g_start(c, b):
        pltpu.make_async_copy(table.at[idx_v.at[c]], rows.at[b], gsems[b]).start()

    def g_wait(b):
        pltpu.make_async_copy(table.at[idx_v.at[0]], rows.at[b], gsems[b]).wait()

    def w_start(c, b):
        pltpu.make_async_copy(
            rows.at[b], out_hbm.at[pl.ds(base + c * _CH, _CH)], wsems[b]
        ).start()

    def w_wait(b):
        pltpu.make_async_copy(
            rows.at[b], out_hbm.at[pl.ds(base, _CH)], wsems[b]
        ).wait()

    for b in range(_NBUF):
        g_start(b, b)

    ngroups = nch // _NBUF

    def group(g, carry):
        for b in range(_NBUF):
            c = g * _NBUF + b
            g_wait(b)
            w_start(c, b)
            w_wait(b)
            g_start(c + _NBUF, b)
        return carry

    lax.fori_loop(0, ngroups - 1, group, 0)
    for b in range(_NBUF):
        c = (ngroups - 1) * _NBUF + b
        g_wait(b)
        w_start(c, b)
        w_wait(b)


def _sc_gather(xi, li, wtab, ltab):
    mesh = plsc.VectorSubcoreMesh(
        core_axis_name="c", subcore_axis_name="s",
        num_cores=_NC, num_subcores=_NS,
    )

    @functools.partial(
        pl.kernel,
        out_type=[
            jax.ShapeDtypeStruct((_B * _LX, _D), jnp.float32),
            jax.ShapeDtypeStruct((_B * _LL, _D), jnp.float32),
        ],
        mesh=mesh,
        scratch_types=[
            pltpu.VMEM((_XCH, _CH), jnp.int32),
            pltpu.VMEM((_NBUF, _CH, _D), jnp.float32),
        ] + [pltpu.SemaphoreType.DMA] * (2 * _NBUF),
        compiler_params=pltpu.CompilerParams(use_tc_tiling_on_sc=False),
    )
    def run(xi_hbm, li_hbm, wtab_hbm, ltab_hbm, xe_hbm, le_hbm, idx_v, rows, *sems):
        wid = lax.axis_index("s") * _NC + lax.axis_index("c")
        gs, ws = sems[:_NBUF], sems[_NBUF:]
        _gather_phase(xi_hbm, wtab_hbm, xe_hbm, idx_v, rows, gs, ws, _XCH, wid)
        _gather_phase(li_hbm, ltab_hbm, le_hbm, idx_v, rows, gs, ws, _LCH, wid)

    return run(xi, li, wtab, ltab)


def _regroup(v, nblk, bt):
    v = v.reshape(bt // 8, nblk, 8, 128)
    return jnp.concatenate(
        [v[:, j].reshape(bt, 128) for j in range(nblk)], axis=1)


def _mlp_body(xe_ref, le_ref, w1x_ref, w1l_ref, b1_ref, w2_ref, b2_ref,
              w3_ref, b3_ref, out_ref):
    bt = out_ref.shape[0]
    x = _regroup(xe_ref[...], _XBLK, bt)
    l = _regroup(le_ref[...], _LBLK, bt)
    h = jnp.dot(x, w1x_ref[...], preferred_element_type=jnp.float32)
    h = h + jnp.dot(l, w1l_ref[...], preferred_element_type=jnp.float32)
    h = jnp.maximum(h + b1_ref[...], 0.0)
    h = jnp.maximum(
        jnp.dot(h, w2_ref[...], preferred_element_type=jnp.float32) + b2_ref[...], 0.0)
    out_ref[...] = jnp.dot(h, w3_ref[...], preferred_element_type=jnp.float32) + b3_ref[...]


def _mlp(xe, le, w1x, w1l, b1, w2, b2, w3, b3):
    bt = 256
    return pl.pallas_call(
        _mlp_body,
        grid=(_B // bt,),
        in_specs=[
            pl.BlockSpec((bt * _XBLK, 128), lambda i: (i, 0)),
            pl.BlockSpec((bt * _LBLK, 128), lambda i: (i, 0)),
            pl.BlockSpec((_LX * _D, 1024), lambda i: (0, 0)),
            pl.BlockSpec((_LL * _D, 1024), lambda i: (0, 0)),
            pl.BlockSpec((1, 1024), lambda i: (0, 0)),
            pl.BlockSpec((1024, 512), lambda i: (0, 0)),
            pl.BlockSpec((1, 512), lambda i: (0, 0)),
            pl.BlockSpec((512, _LX), lambda i: (0, 0)),
            pl.BlockSpec((1, _LX), lambda i: (0, 0)),
        ],
        out_specs=pl.BlockSpec((bt, _LX), lambda i: (i, 0)),
        out_shape=jax.ShapeDtypeStruct((_B, _LX), jnp.float32),
    )(xe, le, w1x, w1l, b1, w2, b2, w3, b3)


def _tile_order(idx, nblk):
    return idx.reshape(_B // 8, 8, nblk, 2).transpose(0, 2, 1, 3).reshape(-1)


def _prep_body(src_ref, out_ref):
    xt = src_ref[...].T
    out_ref[...] = jnp.concatenate([xt, xt], axis=1)


def _prep(t):
    n = t.shape[0]
    nb = 8192
    tT = t.T
    out = pl.pallas_call(
        _prep_body,
        grid=(n // nb,),
        in_specs=[pl.BlockSpec((_D, nb), lambda i: (0, i))],
        out_specs=pl.BlockSpec((nb, 2 * _D), lambda i: (i, 0)),
        out_shape=jax.ShapeDtypeStruct((n, 2 * _D), jnp.float32),
    )(tT)
    return out.reshape(2 * n, _D)


def kernel(x, lemma, word_emb, lemma_emb, W1, b1, W2, b2, W3, b3):
    xi = (2 * _tile_order(x, _XBLK)).reshape(_NW, _XCH, _CH)
    li = (2 * _tile_order(lemma, _LBLK)).reshape(_NW, _LCH, _CH)
    xe, le = _sc_gather(xi, li, _prep(word_emb), _prep(lemma_emb))
    xe = xe.reshape(_B * _LX // 2, 2 * _D)
    le = le.reshape(_B * _LL // 2, 2 * _D)
    return _mlp(xe, le, W1[: _LX * _D], W1[_LX * _D:], b1.reshape(1, -1),
                W2, b2.reshape(1, -1), W3, b3.reshape(1, -1))


# speedup vs baseline: 5.3980x; 1.2552x over previous
import functools

import numpy as np

import jax
import jax.numpy as jnp
from jax import lax
from jax.experimental import pallas as pl
from jax.experimental.pallas import tpu as pltpu
from jax.experimental.pallas import tpu_sc as plsc

_B = 4096
_LX = 50
_LL = 20
_D = 64
_NC, _NS = 2, 16
_NW = _NC * _NS
_CH = 128
_NBUF = 5
_XBLK = _LX // 2
_LBLK = _LL // 2

_XCH = _B * _LX // (_NW * _CH)
_LCH = _B * _LL // (_NW * _CH)


def _gather_phase(idx_hbm, table, out_hbm, idx_v, rows, gsems, wsems, nch, wid):
    pltpu.sync_copy(idx_hbm.at[wid], idx_v.at[pl.ds(0, nch)])
    base = wid * nch * _CH

    def g_start(c, b):
        pltpu.make_async_copy(table.at[idx_v.at[c]], rows.at[b], gsems[b]).start()

    def g_wait(b):
        pltpu.make_async_copy(table.at[idx_v.at[0]], rows.at[b], gsems[b]).wait()

    def w_start(c, b):
        pltpu.make_async_copy(
            rows.at[b], out_hbm.at[pl.ds(base + c * _CH, _CH)], wsems[b]
        ).start()

    def w_wait(b):
        pltpu.make_async_copy(
            rows.at[b], out_hbm.at[pl.ds(base, _CH)], wsems[b]
        ).wait()

    for b in range(_NBUF):
        g_start(b, b)

    ngroups = nch // _NBUF

    def group(g, carry):
        for b in range(_NBUF):
            c = g * _NBUF + b
            g_wait(b)
            w_start(c, b)
            w_wait(b)
            g_start(c + _NBUF, b)
        return carry

    lax.fori_loop(0, ngroups - 1, group, 0)
    for b in range(_NBUF):
        c = (ngroups - 1) * _NBUF + b
        g_wait(b)
        w_start(c, b)
        w_wait(b)


def _sc_gather(xi, li, wtab, ltab):
    mesh = plsc.VectorSubcoreMesh(
        core_axis_name="c", subcore_axis_name="s",
        num_cores=_NC, num_subcores=_NS,
    )

    @functools.partial(
        pl.kernel,
        out_type=[
            jax.ShapeDtypeStruct((_B * _LX, _D), jnp.float32),
            jax.ShapeDtypeStruct((_B * _LL, _D), jnp.float32),
        ],
        mesh=mesh,
        scratch_types=[
            pltpu.VMEM((_XCH, _CH), jnp.int32),
            pltpu.VMEM((_NBUF, _CH, _D), jnp.float32),
        ] + [pltpu.SemaphoreType.DMA] * (2 * _NBUF),
        compiler_params=pltpu.CompilerParams(use_tc_tiling_on_sc=False),
    )
    def run(xi_hbm, li_hbm, wtab_hbm, ltab_hbm, xe_hbm, le_hbm, idx_v, rows, *sems):
        wid = lax.axis_index("s") * _NC + lax.axis_index("c")
        gs, ws = sems[:_NBUF], sems[_NBUF:]
        _gather_phase(xi_hbm, wtab_hbm, xe_hbm, idx_v, rows, gs, ws, _XCH, wid)
        _gather_phase(li_hbm, ltab_hbm, le_hbm, idx_v, rows, gs, ws, _LCH, wid)

    return run(xi, li, wtab, ltab)


def _regroup(v, nblk, bt):
    v = v.reshape(bt // 8, nblk, 8, 128)
    return jnp.concatenate(
        [v[:, j].reshape(bt, 128) for j in range(nblk)], axis=1)


def _mlp_body(xe_ref, le_ref, w1x_ref, w1l_ref, b1_ref, w2_ref, b2_ref,
              w3_ref, b3_ref, out_ref):
    bt = out_ref.shape[0]
    x = _regroup(xe_ref[...], _XBLK, bt)
    l = _regroup(le_ref[...], _LBLK, bt)
    h = jnp.dot(x, w1x_ref[...], preferred_element_type=jnp.float32)
    h = h + jnp.dot(l, w1l_ref[...], preferred_element_type=jnp.float32)
    h = jnp.maximum(h + b1_ref[...], 0.0)
    h = jnp.maximum(
        jnp.dot(h, w2_ref[...], preferred_element_type=jnp.float32) + b2_ref[...], 0.0)
    out_ref[...] = jnp.dot(h, w3_ref[...], preferred_element_type=jnp.float32) + b3_ref[...]


def _mlp(xe, le, w1x, w1l, b1, w2, b2, w3, b3):
    bt = 256
    return pl.pallas_call(
        _mlp_body,
        grid=(_B // bt,),
        in_specs=[
            pl.BlockSpec((bt * _XBLK, 128), lambda i: (i, 0)),
            pl.BlockSpec((bt * _LBLK, 128), lambda i: (i, 0)),
            pl.BlockSpec((_LX * _D, 1024), lambda i: (0, 0)),
            pl.BlockSpec((_LL * _D, 1024), lambda i: (0, 0)),
            pl.BlockSpec((1, 1024), lambda i: (0, 0)),
            pl.BlockSpec((1024, 512), lambda i: (0, 0)),
            pl.BlockSpec((1, 512), lambda i: (0, 0)),
            pl.BlockSpec((512, _LX), lambda i: (0, 0)),
            pl.BlockSpec((1, _LX), lambda i: (0, 0)),
        ],
        out_specs=pl.BlockSpec((bt, _LX), lambda i: (i, 0)),
        out_shape=jax.ShapeDtypeStruct((_B, _LX), jnp.float32),
    )(xe, le, w1x, w1l, b1, w2, b2, w3, b3)


def _perm(nblk):
    u = np.arange(_B * nblk * 2)
    rb, rem = u // (nblk * 16), u % (nblk * 16)
    pblk, r2 = rem // 16, rem % 16
    b = rb * 8 + r2 // 2
    j = 2 * pblk + r2 % 2
    return jnp.asarray(b * (2 * nblk) + j, dtype=jnp.int32)


_PX = _perm(_XBLK)
_PL = _perm(_LBLK)


def _tile_order(idx, nblk):
    return idx.reshape(-1)[_PX if nblk == _XBLK else _PL]


def _prep_body(src_ref, out_ref):
    xt = src_ref[...].T
    out_ref[...] = jnp.concatenate([xt, xt], axis=1)


def _prep(t):
    n = t.shape[0]
    nb = 8192
    tT = t.T
    out = pl.pallas_call(
        _prep_body,
        grid=(n // nb,),
        in_specs=[pl.BlockSpec((_D, nb), lambda i: (0, i))],
        out_specs=pl.BlockSpec((nb, 2 * _D), lambda i: (i, 0)),
        out_shape=jax.ShapeDtypeStruct((n, 2 * _D), jnp.float32),
    )(tT)
    return out.reshape(2 * n, _D)


def kernel(x, lemma, word_emb, lemma_emb, W1, b1, W2, b2, W3, b3):
    xi = (2 * _tile_order(x, _XBLK)).reshape(_NW, _XCH, _CH)
    li = (2 * _tile_order(lemma, _LBLK)).reshape(_NW, _LCH, _CH)
    xe, le = _sc_gather(xi, li, _prep(word_emb), _prep(lemma_emb))
    xe = xe.reshape(_B * _LX // 2, 2 * _D)
    le = le.reshape(_B * _LL // 2, 2 * _D)
    return _mlp(xe, le, W1[: _LX * _D], W1[_LX * _D:], b1.reshape(1, -1),
                W2, b2.reshape(1, -1), W3, b3.reshape(1, -1))
